# baseline (speedup 1.0000x reference)
import jax
import jax.numpy as jnp
from jax import lax
from jax.experimental import pallas as pl
from jax.experimental.pallas import tpu as pltpu
from jax.experimental.pallas import tpu_sc as plsc

C = 65536
P = 65536
NNZ = 262144
NS = 16
NC = 2
LB = 16
NCHUNK = 4
EPT = NNZ // NS
GS = 128
NG = EPT // GS
RING = 4
NBLK = NG // RING
ZROWS = 32
RPT = C // NS
DCH = 128
OBW = 512
PPW = (P // NS) // NC


def _body(table, inp, cols_h, rows_h, vals_h, out,
          cols_v, rows_v, buf, vbuf, zbuf, dbuf, pdbuf, obuf, acc,
          g0, g1, g2, g3, s0, s1, s2, s3, v0, v1, v2, v3, psem):
    core = lax.axis_index("c")
    tile = lax.axis_index("s")
    gsems = (g0, g1, g2, g3)
    ssems = (s0, s1, s2, s3)
    vsems = (v0, v1, v2, v3)
    iota = lax.iota(jnp.int32, LB)
    iota17 = iota * 17

    is_copier = jnp.logical_and(core == 0, tile == 0)

    @pl.when(is_copier)
    def _():
        pltpu.async_copy(inp, out.at[:, pl.ds(0, P)], psem)

    pltpu.sync_copy(cols_h.at[tile], cols_v)
    pltpu.sync_copy(rows_h.at[tile], rows_v)

    def zb_body(i, c_):
        zbuf[i, :] = jnp.zeros((LB,), jnp.float32)
        return c_
    lax.fori_loop(0, ZROWS, zb_body, 0)

    def adjust_cols(delta):
        def adj_body(i, c_):
            sl = pl.ds(i * LB, LB)
            cols_v[sl] = cols_v[sl] + delta
            return c_
        lax.fori_loop(0, EPT // LB, adj_body, 0)

    def fire_gather(g, j):
        pltpu.async_copy(table.at[cols_v.at[pl.ds(g * GS, GS)]],
                         buf.at[j], gsems[j])
        pltpu.async_copy(vals_h.at[tile, g], vbuf.at[j], vsems[j])

    def wait_gather(j):
        pltpu.make_async_copy(table.at[pl.ds(0, GS)], buf.at[j],
                              gsems[j]).wait()
        pltpu.make_async_copy(vals_h.at[0, 0], vbuf.at[j], vsems[j]).wait()

    def fire_scatter(g, j):
        pltpu.async_copy(buf.at[j], acc.at[rows_v.at[g]], ssems[j], add=True)

    def wait_scatter(j):
        pltpu.make_async_copy(table.at[pl.ds(0, GS)], buf.at[j],
                              ssems[j]).wait()

    def scale(j):
        @plsc.parallel_loop(0, GS // LB, unroll=2)
        def sc_body(sv):
            vv = vbuf[j, pl.ds(sv * LB, LB)]
            for k in range(LB):
                e = sv * LB + k
                buf[j, e, :] = buf[j, e, :] * vv[k]

    for p in range(2):
        if p == 0:
            adjust_cols(core * (2 * P))
        else:
            adjust_cols(P)
        chunk = core * 2 + p

        for zi in range(RPT // ZROWS):
            pltpu.sync_copy(zbuf, acc.at[pl.ds(tile * RPT + zi * ZROWS,
                                               ZROWS)])
        plsc.subcore_barrier()

        for j in range(RING):
            fire_gather(j, j)

        def blk_body(blk, c_):
            for j in range(RING):
                wait_gather(j)
                scale(j)
                fire_scatter(blk * RING + j, j)
            for j in range(RING):
                wait_scatter(j)

            @pl.when(blk + 1 < NBLK)
            def _():
                for j in range(RING):
                    fire_gather((blk + 1) * RING + j, j)
            return c_
        lax.fori_loop(0, NBLK, blk_body, 0)

        plsc.subcore_barrier()

        def og_body(og, c_):
            def ch_body(ch, c2_):
                r0 = tile * RPT + og * OBW + ch * DCH
                pltpu.sync_copy(acc.at[pl.ds(r0, DCH)], dbuf)

                @plsc.parallel_loop(0, DCH, unroll=4)
                def rp_body(i):
                    plsc.store_scatter(pdbuf, [iota + i * 17], dbuf[i, :])
                for jj in range(LB):
                    for c16 in range(DCH // LB):
                        v = plsc.load_gather(pdbuf,
                                             [iota17 + (272 * c16 + jj)])
                        obuf[jj, pl.ds(ch * DCH + c16 * LB, LB)] = v
                return c2_
            lax.fori_loop(0, OBW // DCH, ch_body, 0)
            pltpu.sync_copy(
                obuf,
                out.at[pl.ds(chunk * LB, LB),
                       pl.ds(P + tile * RPT + og * OBW, OBW)])
            return c_
        lax.fori_loop(0, RPT // OBW, og_body, 0)
        plsc.subcore_barrier()

    @pl.when(is_copier)
    def _():
        pltpu.make_async_copy(inp, out.at[:, pl.ds(0, P)], psem).wait()


_sc_call = pl.kernel(
    _body,
    out_type=jax.ShapeDtypeStruct((NCHUNK * LB, P + C), jnp.float32),
    mesh=plsc.VectorSubcoreMesh(core_axis_name="c", subcore_axis_name="s"),
    scratch_types=[
        pltpu.VMEM((EPT,), jnp.int32),
        pltpu.VMEM((NG, GS), jnp.int32),
        pltpu.VMEM((RING, GS, LB), jnp.float32),
        pltpu.VMEM((RING, GS), jnp.float32),
        pltpu.VMEM((ZROWS, LB), jnp.float32),
        pltpu.VMEM((DCH, LB), jnp.float32),
        pltpu.VMEM((DCH * 17,), jnp.float32),
        pltpu.VMEM((LB, OBW), jnp.float32),
        pltpu.VMEM_SHARED((C, LB), jnp.float32),
        pltpu.SemaphoreType.DMA,
        pltpu.SemaphoreType.DMA,
        pltpu.SemaphoreType.DMA,
        pltpu.SemaphoreType.DMA,
        pltpu.SemaphoreType.DMA,
        pltpu.SemaphoreType.DMA,
        pltpu.SemaphoreType.DMA,
        pltpu.SemaphoreType.DMA,
        pltpu.SemaphoreType.DMA,
        pltpu.SemaphoreType.DMA,
        pltpu.SemaphoreType.DMA,
        pltpu.SemaphoreType.DMA,
        pltpu.SemaphoreType.DMA,
    ],
    compiler_params=pltpu.CompilerParams(use_tc_tiling_on_sc=False,
                                         needs_layout_passes=False),
)


@jax.jit
def kernel(input, weight_indices, weight_values):
    rows = weight_indices[0].astype(jnp.int32)
    cols = weight_indices[1].astype(jnp.int32)
    table = input.reshape(NCHUNK, LB, P).transpose(0, 2, 1).reshape(
        NCHUNK * P, LB)
    return _sc_call(
        table,
        input,
        cols.reshape(NS, EPT),
        rows.reshape(NS, NG, GS),
        weight_values.reshape(NS, NG, GS),
    )


# speedup vs baseline: 1.8145x; 1.8145x over previous
import jax
import jax.numpy as jnp
from jax import lax
from jax.experimental import pallas as pl
from jax.experimental.pallas import tpu as pltpu
from jax.experimental.pallas import tpu_sc as plsc

C = 65536
P = 65536
NNZ = 262144
NS = 16
NC = 2
LB = 16
NCHUNK = 4
EPT = NNZ // NS
GS = 128
NG = EPT // GS
RING = 4
NBLK = NG // RING
ZROWS = 32
RPT = C // NS
DCH = 128
OBW = 512
PPW = (P // NS) // NC


def _body(table, inp, cols_h, rows_h, vals_h, out,
          cols_v, rows_v, buf, vbuf, zbuf, dbuf, pdbuf, obuf, acc,
          g0, g1, g2, g3, s0, s1, s2, s3, v0, v1, v2, v3, psem):
    core = lax.axis_index("c")
    tile = lax.axis_index("s")
    gsems = (g0, g1, g2, g3)
    ssems = (s0, s1, s2, s3)
    vsems = (v0, v1, v2, v3)
    iota = lax.iota(jnp.int32, LB)
    iota17 = iota * 17

    w = core * NS + tile
    for jg in range(4):
        for cb in range(PPW // OBW):
            csl = pl.ds(w * PPW + cb * OBW, OBW)
            pltpu.sync_copy(inp.at[pl.ds(jg * LB, LB), csl], obuf)
            pltpu.sync_copy(obuf, out.at[pl.ds(jg * LB, LB), csl])

    pltpu.sync_copy(cols_h.at[tile], cols_v)
    pltpu.sync_copy(rows_h.at[tile], rows_v)

    def zb_body(i, c_):
        zbuf[i, :] = jnp.zeros((LB,), jnp.float32)
        return c_
    lax.fori_loop(0, ZROWS, zb_body, 0)

    def adjust_cols(delta):
        def adj_body(i, c_):
            sl = pl.ds(i * LB, LB)
            cols_v[sl] = cols_v[sl] + delta
            return c_
        lax.fori_loop(0, EPT // LB, adj_body, 0)

    def fire_gather(g, j):
        pltpu.async_copy(table.at[cols_v.at[pl.ds(g * GS, GS)]],
                         buf.at[j], gsems[j])
        pltpu.async_copy(vals_h.at[tile, g], vbuf.at[j], vsems[j])

    def wait_gather(j):
        pltpu.make_async_copy(table.at[pl.ds(0, GS)], buf.at[j],
                              gsems[j]).wait()
        pltpu.make_async_copy(vals_h.at[0, 0], vbuf.at[j], vsems[j]).wait()

    def fire_scatter(g, j):
        pltpu.async_copy(buf.at[j], acc.at[rows_v.at[g]], ssems[j], add=True)

    def wait_scatter(j):
        pltpu.make_async_copy(table.at[pl.ds(0, GS)], buf.at[j],
                              ssems[j]).wait()

    def scale(j):
        @plsc.parallel_loop(0, GS // LB, unroll=2)
        def sc_body(sv):
            vv = vbuf[j, pl.ds(sv * LB, LB)]
            for k in range(LB):
                e = sv * LB + k
                buf[j, e, :] = buf[j, e, :] * vv[k]

    for p in range(2):
        if p == 0:
            adjust_cols(core * (2 * P))
        else:
            adjust_cols(P)
        chunk = core * 2 + p

        for zi in range(RPT // ZROWS):
            pltpu.sync_copy(zbuf, acc.at[pl.ds(tile * RPT + zi * ZROWS,
                                               ZROWS)])
        plsc.subcore_barrier()

        for j in range(RING):
            fire_gather(j, j)

        def blk_body(blk, c_):
            for j in range(RING):
                wait_gather(j)
                scale(j)
                fire_scatter(blk * RING + j, j)
            for j in range(RING):
                wait_scatter(j)

            @pl.when(blk + 1 < NBLK)
            def _():
                for j in range(RING):
                    fire_gather((blk + 1) * RING + j, j)
            return c_
        lax.fori_loop(0, NBLK, blk_body, 0)

        plsc.subcore_barrier()

        def og_body(og, c_):
            def ch_body(ch, c2_):
                r0 = tile * RPT + og * OBW + ch * DCH
                pltpu.sync_copy(acc.at[pl.ds(r0, DCH)], dbuf)

                @plsc.parallel_loop(0, DCH, unroll=4)
                def rp_body(i):
                    plsc.store_scatter(pdbuf, [iota + i * 17], dbuf[i, :])
                for jj in range(LB):
                    for c16 in range(DCH // LB):
                        v = plsc.load_gather(pdbuf,
                                             [iota17 + (272 * c16 + jj)])
                        obuf[jj, pl.ds(ch * DCH + c16 * LB, LB)] = v
                return c2_
            lax.fori_loop(0, OBW // DCH, ch_body, 0)
            pltpu.sync_copy(
                obuf,
                out.at[pl.ds(chunk * LB, LB),
                       pl.ds(P + tile * RPT + og * OBW, OBW)])
            return c_
        lax.fori_loop(0, RPT // OBW, og_body, 0)
        plsc.subcore_barrier()


_sc_call = pl.kernel(
    _body,
    out_type=jax.ShapeDtypeStruct((NCHUNK * LB, P + C), jnp.float32),
    mesh=plsc.VectorSubcoreMesh(core_axis_name="c", subcore_axis_name="s"),
    scratch_types=[
        pltpu.VMEM((EPT,), jnp.int32),
        pltpu.VMEM((NG, GS), jnp.int32),
        pltpu.VMEM((RING, GS, LB), jnp.float32),
        pltpu.VMEM((RING, GS), jnp.float32),
        pltpu.VMEM((ZROWS, LB), jnp.float32),
        pltpu.VMEM((DCH, LB), jnp.float32),
        pltpu.VMEM((DCH * 17,), jnp.float32),
        pltpu.VMEM((LB, OBW), jnp.float32),
        pltpu.VMEM_SHARED((C, LB), jnp.float32),
        pltpu.SemaphoreType.DMA,
        pltpu.SemaphoreType.DMA,
        pltpu.SemaphoreType.DMA,
        pltpu.SemaphoreType.DMA,
        pltpu.SemaphoreType.DMA,
        pltpu.SemaphoreType.DMA,
        pltpu.SemaphoreType.DMA,
        pltpu.SemaphoreType.DMA,
        pltpu.SemaphoreType.DMA,
        pltpu.SemaphoreType.DMA,
        pltpu.SemaphoreType.DMA,
        pltpu.SemaphoreType.DMA,
        pltpu.SemaphoreType.DMA,
    ],
    compiler_params=pltpu.CompilerParams(use_tc_tiling_on_sc=False,
                                         needs_layout_passes=False),
)


@jax.jit
def kernel(input, weight_indices, weight_values):
    rows = weight_indices[0].astype(jnp.int32)
    cols = weight_indices[1].astype(jnp.int32)
    table = input.reshape(NCHUNK, LB, P).transpose(0, 2, 1).reshape(
        NCHUNK * P, LB)
    return _sc_call(
        table,
        input,
        cols.reshape(NS, EPT),
        rows.reshape(NS, NG, GS),
        weight_values.reshape(NS, NG, GS),
    )


# speedup vs baseline: 1.8983x; 1.0462x over previous
import jax
import jax.numpy as jnp
from jax import lax
from jax.experimental import pallas as pl
from jax.experimental.pallas import tpu as pltpu
from jax.experimental.pallas import tpu_sc as plsc

C = 65536
P = 65536
NNZ = 262144
NS = 16
NC = 2
LB = 16
NCHUNK = 4
EPT = NNZ // NS
GS = 128
NG = EPT // GS
RING = 4
NBLK = NG // RING
ZROWS = 32
RPT = C // NS
DCH = 128
OBW = 512
PPW = (P // NS) // NC


def _body(inp, cols_h, rows_h, vals_h, out, table,
          cols_v, rows_v, buf, vbuf, zbuf, dbuf, pdbuf, obuf, acc,
          g0, g1, g2, g3, s0, s1, s2, s3, v0, v1, v2, v3, psem):
    core = lax.axis_index("c")
    tile = lax.axis_index("s")
    gsems = (g0, g1, g2, g3)
    ssems = (s0, s1, s2, s3)
    vsems = (v0, v1, v2, v3)
    iota = lax.iota(jnp.int32, LB)
    iota17 = iota * 17
    iota129 = iota * 129

    w = core * NS + tile
    for jg in range(4):
        for cb in range(PPW // OBW):
            csl = pl.ds(w * PPW + cb * OBW, OBW)
            pltpu.sync_copy(inp.at[pl.ds(jg * LB, LB), csl], obuf)
            pltpu.sync_copy(obuf, out.at[pl.ds(jg * LB, LB), csl])

    pltpu.sync_copy(cols_h.at[tile], cols_v)
    pltpu.sync_copy(rows_h.at[tile], rows_v)

    for bp in range(2):
        bchunk = core * 2 + bp

        def bb(sb, c_):
            pbase = tile * RPT + sb * DCH
            pltpu.sync_copy(inp.at[pl.ds(bchunk * LB, LB), pl.ds(pbase, DCH)],
                            obuf.at[:, pl.ds(0, DCH)])

            @plsc.parallel_loop(0, LB, unroll=2)
            def rp(jj):
                for k in range(DCH // LB):
                    v = obuf[jj, pl.ds(k * LB, LB)]
                    plsc.store_scatter(pdbuf, [iota + (jj * 129 + k * LB)], v)

            @plsc.parallel_loop(0, DCH, unroll=4)
            def tr(i):
                dbuf[i, :] = plsc.load_gather(pdbuf, [iota129 + i])

            pltpu.sync_copy(dbuf,
                            table.at[pl.ds(bchunk * P + pbase, DCH)])
            return c_
        lax.fori_loop(0, RPT // DCH, bb, 0)

    def zb_body(i, c_):
        zbuf[i, :] = jnp.zeros((LB,), jnp.float32)
        return c_
    lax.fori_loop(0, ZROWS, zb_body, 0)

    def adjust_cols(delta):
        def adj_body(i, c_):
            sl = pl.ds(i * LB, LB)
            cols_v[sl] = cols_v[sl] + delta
            return c_
        lax.fori_loop(0, EPT // LB, adj_body, 0)

    def fire_gather(g, j):
        pltpu.async_copy(table.at[cols_v.at[pl.ds(g * GS, GS)]],
                         buf.at[j], gsems[j])
        pltpu.async_copy(vals_h.at[tile, g], vbuf.at[j], vsems[j])

    def wait_gather(j):
        pltpu.make_async_copy(table.at[pl.ds(0, GS)], buf.at[j],
                              gsems[j]).wait()
        pltpu.make_async_copy(vals_h.at[0, 0], vbuf.at[j], vsems[j]).wait()

    def fire_scatter(g, j):
        pltpu.async_copy(buf.at[j], acc.at[rows_v.at[g]], ssems[j], add=True)

    def wait_scatter(j):
        pltpu.make_async_copy(table.at[pl.ds(0, GS)], buf.at[j],
                              ssems[j]).wait()

    def scale(j):
        @plsc.parallel_loop(0, GS // LB, unroll=2)
        def sc_body(sv):
            vv = vbuf[j, pl.ds(sv * LB, LB)]
            for k in range(LB):
                e = sv * LB + k
                buf[j, e, :] = buf[j, e, :] * vv[k]

    for p in range(2):
        if p == 0:
            adjust_cols(core * (2 * P))
        else:
            adjust_cols(P)
        chunk = core * 2 + p

        for zi in range(RPT // ZROWS):
            pltpu.sync_copy(zbuf, acc.at[pl.ds(tile * RPT + zi * ZROWS,
                                               ZROWS)])
        plsc.subcore_barrier()

        for j in range(RING):
            fire_gather(j, j)

        def blk_body(blk, c_):
            for j in range(RING):
                wait_gather(j)
                scale(j)
                fire_scatter(blk * RING + j, j)
            for j in range(RING):
                wait_scatter(j)

            @pl.when(blk + 1 < NBLK)
            def _():
                for j in range(RING):
                    fire_gather((blk + 1) * RING + j, j)
            return c_
        lax.fori_loop(0, NBLK, blk_body, 0)

        plsc.subcore_barrier()

        def og_body(og, c_):
            def ch_body(ch, c2_):
                r0 = tile * RPT + og * OBW + ch * DCH
                pltpu.sync_copy(acc.at[pl.ds(r0, DCH)], dbuf)

                @plsc.parallel_loop(0, DCH, unroll=4)
                def rp_body(i):
                    plsc.store_scatter(pdbuf, [iota + i * 17], dbuf[i, :])
                for jj in range(LB):
                    for c16 in range(DCH // LB):
                        v = plsc.load_gather(pdbuf,
                                             [iota17 + (272 * c16 + jj)])
                        obuf[jj, pl.ds(ch * DCH + c16 * LB, LB)] = v
                return c2_
            lax.fori_loop(0, OBW // DCH, ch_body, 0)
            pltpu.sync_copy(
                obuf,
                out.at[pl.ds(chunk * LB, LB),
                       pl.ds(P + tile * RPT + og * OBW, OBW)])
            return c_
        lax.fori_loop(0, RPT // OBW, og_body, 0)
        plsc.subcore_barrier()


_sc_call = pl.kernel(
    _body,
    out_type=[jax.ShapeDtypeStruct((NCHUNK * LB, P + C), jnp.float32),
              jax.ShapeDtypeStruct((NCHUNK * P, LB), jnp.float32)],
    mesh=plsc.VectorSubcoreMesh(core_axis_name="c", subcore_axis_name="s"),
    scratch_types=[
        pltpu.VMEM((EPT,), jnp.int32),
        pltpu.VMEM((NG, GS), jnp.int32),
        pltpu.VMEM((RING, GS, LB), jnp.float32),
        pltpu.VMEM((RING, GS), jnp.float32),
        pltpu.VMEM((ZROWS, LB), jnp.float32),
        pltpu.VMEM((DCH, LB), jnp.float32),
        pltpu.VMEM((DCH * 17,), jnp.float32),
        pltpu.VMEM((LB, OBW), jnp.float32),
        pltpu.VMEM_SHARED((C, LB), jnp.float32),
        pltpu.SemaphoreType.DMA,
        pltpu.SemaphoreType.DMA,
        pltpu.SemaphoreType.DMA,
        pltpu.SemaphoreType.DMA,
        pltpu.SemaphoreType.DMA,
        pltpu.SemaphoreType.DMA,
        pltpu.SemaphoreType.DMA,
        pltpu.SemaphoreType.DMA,
        pltpu.SemaphoreType.DMA,
        pltpu.SemaphoreType.DMA,
        pltpu.SemaphoreType.DMA,
        pltpu.SemaphoreType.DMA,
        pltpu.SemaphoreType.DMA,
    ],
    compiler_params=pltpu.CompilerParams(use_tc_tiling_on_sc=False,
                                         needs_layout_passes=False),
)


@jax.jit
def kernel(input, weight_indices, weight_values):
    rows = weight_indices[0].astype(jnp.int32)
    cols = weight_indices[1].astype(jnp.int32)
    out, _table = _sc_call(
        input,
        cols.reshape(NS, EPT),
        rows.reshape(NS, NG, GS),
        weight_values.reshape(NS, NG, GS),
    )
    return out
